# initial kernel scaffold (unmeasured)
import jax
import jax.numpy as jnp
from jax import lax
from jax.experimental import pallas as pl
from jax.experimental.pallas import tpu as pltpu


def kernel(
    x,
):
    def body(*refs):
        pass

    out_shape = jax.ShapeDtypeStruct(..., jnp.float32)
    return pl.pallas_call(body, out_shape=out_shape)(...)


# baseline (device time: 528522 ns/iter reference)
import functools

import jax
import jax.numpy as jnp
from jax import lax
from jax.experimental import pallas as pl
from jax.experimental.pallas import tpu as pltpu

K = 32
NZ = 4
BLK = 128
NEG = float("-inf")


def _topk_with_counts(x, nrows, reduce_first_axis):
    lane = lax.broadcasted_iota(jnp.int32, (nrows, K), 1)

    def it(k, carry):
        xv, vals, cnts = carry
        if reduce_first_axis:
            mx = jnp.max(jnp.max(xv, axis=2), axis=0)
            mxb = mx[None, :, None]
        else:
            mx = jnp.max(xv, axis=1)
            mxb = mx[:, None]
        eq = xv == mxb
        eqf = eq.astype(jnp.float32)
        if reduce_first_axis:
            cnt = jnp.sum(jnp.sum(eqf, axis=2), axis=0)
        else:
            cnt = jnp.sum(eqf, axis=1)
        vals = jnp.where(lane == k, mx[:, None], vals)
        cnts = jnp.where(lane == k, cnt[:, None], cnts)
        xv = jnp.where(eq, NEG, xv)
        return xv, vals, cnts

    vals0 = jnp.full((nrows, K), NEG, jnp.float32)
    cnts0 = jnp.zeros((nrows, K), jnp.float32)
    _, vals, cnts = lax.fori_loop(0, K, it, (x, vals0, cnts0))
    return vals, cnts


def _expand_sorted(vals, cnts, nrows):
    ii = lax.broadcasted_iota(jnp.int32, (K, K), 0)
    jj = lax.broadcasted_iota(jnp.int32, (K, K), 1)
    tri = (ii <= jj).astype(jnp.float32)
    cum = jnp.dot(cnts, tri, preferred_element_type=jnp.float32)
    jidx = lax.broadcasted_iota(jnp.int32, (nrows, K, K), 2).astype(jnp.float32)
    mask = cum[:, :, None] > jidx
    return jnp.max(jnp.where(mask, vals[:, :, None], NEG), axis=1)


def kernel(x):
    m, n = x.shape
    nblk = m // BLK

    def body(x_ref, out_ref, comm_ref, send_sems, recv_sems):
        step = pl.program_id(0)
        my_x = lax.axis_index("x")
        my_y = lax.axis_index("y")
        my_z = lax.axis_index("z")

        vals, cnts = _topk_with_counts(x_ref[...], BLK, False)
        comm_ref[my_z, pl.ds(step * BLK, BLK), :] = _expand_sorted(
            vals, cnts, BLK
        )

        @pl.when(step == nblk - 1)
        def _():
            barrier = pltpu.get_barrier_semaphore()
            for dz in (1, 2, 3):
                tz = lax.rem(my_z + dz, NZ)
                pl.semaphore_signal(
                    barrier, inc=1,
                    device_id=(my_x, my_y, tz),
                    device_id_type=pl.DeviceIdType.MESH,
                )
            pl.semaphore_wait(barrier, 3)

            sends = []
            for dz in (1, 2, 3):
                tz = lax.rem(my_z + dz, NZ)
                rdma = pltpu.make_async_remote_copy(
                    src_ref=comm_ref.at[my_z],
                    dst_ref=comm_ref.at[my_z],
                    send_sem=send_sems.at[dz - 1],
                    recv_sem=recv_sems.at[my_z],
                    device_id=(my_x, my_y, tz),
                    device_id_type=pl.DeviceIdType.MESH,
                )
                rdma.start()
                sends.append(rdma)
            for rdma in sends:
                rdma.wait_send()
            for dz in (1, 2, 3):
                sz = lax.rem(my_z + dz, NZ)
                recv = pltpu.make_async_remote_copy(
                    src_ref=comm_ref.at[my_z],
                    dst_ref=comm_ref.at[sz],
                    send_sem=send_sems.at[0],
                    recv_sem=recv_sems.at[sz],
                    device_id=(my_x, my_y, sz),
                    device_id_type=pl.DeviceIdType.MESH,
                )
                recv.wait_recv()

            gvals, gcnts = _topk_with_counts(comm_ref[...], m, True)
            out_ref[...] = _expand_sorted(gvals, gcnts, m)

    return pl.pallas_call(
        body,
        grid=(nblk,),
        in_specs=[
            pl.BlockSpec((BLK, n), lambda i: (i, 0)),
        ],
        out_specs=pl.BlockSpec((m, K), lambda i: (0, 0)),
        out_shape=jax.ShapeDtypeStruct((m, K), jnp.float32),
        scratch_shapes=[
            pltpu.VMEM((NZ, m, K), jnp.float32),
            pltpu.SemaphoreType.DMA((3,)),
            pltpu.SemaphoreType.DMA((NZ,)),
        ],
        compiler_params=pltpu.CompilerParams(
            collective_id=0,
            dimension_semantics=("arbitrary",),
            vmem_limit_bytes=64 * 1024 * 1024,
        ),
    )(x)


# device time: 278018 ns/iter; 1.9010x vs baseline; 1.9010x over previous
import jax
import jax.numpy as jnp
from jax import lax
from jax.experimental import pallas as pl
from jax.experimental.pallas import tpu as pltpu

K = 32
NZ = 4
BLK = 256
IMIN = -(2**31)
IMAX = 2**31 - 1
SIGN = -(2**31)
LOWMASK = -(2**15)


def _encode(x, my_z):
    bits = lax.bitcast_convert_type(x, jnp.int32)
    k2 = jnp.where(bits < 0, ~bits ^ SIGN, bits)
    col = lax.broadcasted_iota(jnp.int32, x.shape, 1)
    return (k2 & LOWMASK) | (my_z * x.shape[1] + col)


def _decode(key):
    kc = key & LOWMASK
    bits = jnp.where(kc < 0, ~(kc ^ SIGN), kc)
    return lax.bitcast_convert_type(bits, jnp.float32)


def _topk_chain(keys, nrows, reduce_first_axis):
    lane = lax.broadcasted_iota(jnp.int32, (nrows, K), 1)

    def it(k, carry):
        m_prev, vals = carry
        if reduce_first_axis:
            cand = jnp.where(keys < m_prev[None, :, None], keys, IMIN)
            m = jnp.max(jnp.max(cand, axis=2), axis=0)
        else:
            cand = jnp.where(keys < m_prev[:, None], keys, IMIN)
            m = jnp.max(cand, axis=1)
        vals = jnp.where(lane == k, m[:, None], vals)
        return m, vals

    m0 = jnp.full((nrows,), IMAX, jnp.int32)
    vals0 = jnp.full((nrows, K), IMIN, jnp.int32)
    _, vals = lax.fori_loop(0, K, it, (m0, vals0))
    return vals


def kernel(x):
    m, n = x.shape
    nblk = m // BLK

    def body(x_ref, out_ref, comm_ref, send_sems, recv_sems):
        step = pl.program_id(0)
        my_x = lax.axis_index("x")
        my_y = lax.axis_index("y")
        my_z = lax.axis_index("z")

        keys = _encode(x_ref[...], my_z)
        comm_ref[my_z, pl.ds(step * BLK, BLK), :] = _topk_chain(
            keys, BLK, False
        )

        @pl.when(step == nblk - 1)
        def _():
            barrier = pltpu.get_barrier_semaphore()
            for dz in (1, 2, 3):
                tz = lax.rem(my_z + dz, NZ)
                pl.semaphore_signal(
                    barrier, inc=1,
                    device_id=(my_x, my_y, tz),
                    device_id_type=pl.DeviceIdType.MESH,
                )
            pl.semaphore_wait(barrier, 3)

            sends = []
            for dz in (1, 2, 3):
                tz = lax.rem(my_z + dz, NZ)
                rdma = pltpu.make_async_remote_copy(
                    src_ref=comm_ref.at[my_z],
                    dst_ref=comm_ref.at[my_z],
                    send_sem=send_sems.at[dz - 1],
                    recv_sem=recv_sems.at[my_z],
                    device_id=(my_x, my_y, tz),
                    device_id_type=pl.DeviceIdType.MESH,
                )
                rdma.start()
                sends.append(rdma)
            for rdma in sends:
                rdma.wait_send()
            for dz in (1, 2, 3):
                sz = lax.rem(my_z + dz, NZ)
                recv = pltpu.make_async_remote_copy(
                    src_ref=comm_ref.at[my_z],
                    dst_ref=comm_ref.at[sz],
                    send_sem=send_sems.at[0],
                    recv_sem=recv_sems.at[sz],
                    device_id=(my_x, my_y, sz),
                    device_id_type=pl.DeviceIdType.MESH,
                )
                recv.wait_recv()

            out_ref[...] = _decode(_topk_chain(comm_ref[...], m, True))

    return pl.pallas_call(
        body,
        grid=(nblk,),
        in_specs=[
            pl.BlockSpec((BLK, n), lambda i: (i, 0)),
        ],
        out_specs=pl.BlockSpec((m, K), lambda i: (0, 0)),
        out_shape=jax.ShapeDtypeStruct((m, K), jnp.float32),
        scratch_shapes=[
            pltpu.VMEM((NZ, m, K), jnp.int32),
            pltpu.SemaphoreType.DMA((3,)),
            pltpu.SemaphoreType.DMA((NZ,)),
        ],
        compiler_params=pltpu.CompilerParams(
            collective_id=0,
            dimension_semantics=("arbitrary",),
            vmem_limit_bytes=64 * 1024 * 1024,
        ),
    )(x)


# device time: 269277 ns/iter; 1.9627x vs baseline; 1.0325x over previous
import jax
import jax.numpy as jnp
from jax import lax
from jax.experimental import pallas as pl
from jax.experimental.pallas import tpu as pltpu

K = 32
NZ = 4
BLK = 256
IMIN = -(2**31)
IMAX = 2**31 - 1
SIGN = -(2**31)
LOWMASK = -(2**15)


def _encode(x, my_z):
    bits = lax.bitcast_convert_type(x, jnp.int32)
    k2 = jnp.where(bits < 0, ~bits ^ SIGN, bits)
    col = lax.broadcasted_iota(jnp.int32, x.shape, 1)
    return (k2 & LOWMASK) | (my_z * x.shape[1] + col)


def _decode(key):
    kc = key & LOWMASK
    bits = jnp.where(kc < 0, ~(kc ^ SIGN), kc)
    return lax.bitcast_convert_type(bits, jnp.float32)


def _topk_chain(keys, nrows, reduce_first_axis, k=K):
    if k == 1 and not reduce_first_axis:
        return jnp.max(keys, axis=1)[:, None]
    lane = lax.broadcasted_iota(jnp.int32, (nrows, k), 1)

    def it(j, carry):
        m_prev, vals = carry
        if reduce_first_axis:
            cand = jnp.where(keys < m_prev[None, :, None], keys, IMIN)
            m = jnp.max(jnp.max(cand, axis=2), axis=0)
        else:
            cand = jnp.where(keys < m_prev[:, None], keys, IMIN)
            m = jnp.max(cand, axis=1)
        vals = jnp.where(lane == j, m[:, None], vals)
        return m, vals

    m0 = jnp.full((nrows,), IMAX, jnp.int32)
    vals0 = jnp.full((nrows, k), IMIN, jnp.int32)
    _, vals = lax.fori_loop(0, k, it, (m0, vals0))
    return vals


def _prune(arr, k, depth):
    if depth == 0 or k == 1:
        return [(arr, k)]
    half = arr.shape[1] // 2
    a = arr[:, :half]
    b = arr[:, half:]
    return _prune(jnp.maximum(a, b), k, depth - 1) + _prune(
        jnp.minimum(a, b), k // 2, depth - 1
    )


def _local_topk(keys, nrows):
    leaves = _prune(keys, K, 3)
    cands = [_topk_chain(arr, nrows, False, k) for arr, k in leaves]
    return _topk_chain(jnp.concatenate(cands, axis=1), nrows, False, K)


def kernel(x):
    m, n = x.shape
    nblk = m // BLK

    def body(x_ref, out_ref, comm_ref, send_sems, recv_sems):
        step = pl.program_id(0)
        my_x = lax.axis_index("x")
        my_y = lax.axis_index("y")
        my_z = lax.axis_index("z")

        keys = _encode(x_ref[...], my_z)
        comm_ref[my_z, pl.ds(step * BLK, BLK), :] = _local_topk(keys, BLK)

        @pl.when(step == nblk - 1)
        def _():
            barrier = pltpu.get_barrier_semaphore()
            for dz in (1, 2, 3):
                tz = lax.rem(my_z + dz, NZ)
                pl.semaphore_signal(
                    barrier, inc=1,
                    device_id=(my_x, my_y, tz),
                    device_id_type=pl.DeviceIdType.MESH,
                )
            pl.semaphore_wait(barrier, 3)

            sends = []
            for dz in (1, 2, 3):
                tz = lax.rem(my_z + dz, NZ)
                rdma = pltpu.make_async_remote_copy(
                    src_ref=comm_ref.at[my_z],
                    dst_ref=comm_ref.at[my_z],
                    send_sem=send_sems.at[dz - 1],
                    recv_sem=recv_sems.at[my_z],
                    device_id=(my_x, my_y, tz),
                    device_id_type=pl.DeviceIdType.MESH,
                )
                rdma.start()
                sends.append(rdma)
            for rdma in sends:
                rdma.wait_send()
            for dz in (1, 2, 3):
                sz = lax.rem(my_z + dz, NZ)
                recv = pltpu.make_async_remote_copy(
                    src_ref=comm_ref.at[my_z],
                    dst_ref=comm_ref.at[sz],
                    send_sem=send_sems.at[0],
                    recv_sem=recv_sems.at[sz],
                    device_id=(my_x, my_y, sz),
                    device_id_type=pl.DeviceIdType.MESH,
                )
                recv.wait_recv()

            out_ref[...] = _decode(_topk_chain(comm_ref[...], m, True))

    return pl.pallas_call(
        body,
        grid=(nblk,),
        in_specs=[
            pl.BlockSpec((BLK, n), lambda i: (i, 0)),
        ],
        out_specs=pl.BlockSpec((m, K), lambda i: (0, 0)),
        out_shape=jax.ShapeDtypeStruct((m, K), jnp.float32),
        scratch_shapes=[
            pltpu.VMEM((NZ, m, K), jnp.int32),
            pltpu.SemaphoreType.DMA((3,)),
            pltpu.SemaphoreType.DMA((NZ,)),
        ],
        compiler_params=pltpu.CompilerParams(
            collective_id=0,
            dimension_semantics=("arbitrary",),
            vmem_limit_bytes=64 * 1024 * 1024,
        ),
    )(x)


# device time: 77063 ns/iter; 6.8583x vs baseline; 3.4942x over previous
import jax
import jax.numpy as jnp
from jax import lax
from jax.experimental import pallas as pl
from jax.experimental.pallas import tpu as pltpu

K = 32
NZ = 4
NQ = 4
IMIN = -(2**31)
IMAX = 2**31 - 1
SIGN = -(2**31)
LOWMASK = -(2**15)


def _encode(x, my_z):
    bits = lax.bitcast_convert_type(x, jnp.int32)
    k2 = jnp.where(bits < 0, ~bits ^ SIGN, bits)
    col = lax.broadcasted_iota(jnp.int32, x.shape, 1)
    return (k2 & LOWMASK) | (my_z * x.shape[1] + col)


def _decode(key):
    kc = key & LOWMASK
    bits = jnp.where(kc < 0, ~(kc ^ SIGN), kc)
    return lax.bitcast_convert_type(bits, jnp.float32)


def _topk_chain(keys, nrows, reduce_first_axis, k=K):
    lane = lax.broadcasted_iota(jnp.int32, (nrows, k), 1)

    def it(j, carry):
        m_prev, vals = carry
        if reduce_first_axis:
            cand = jnp.where(keys < m_prev[None, :, None], keys, IMIN)
            m = jnp.max(jnp.max(cand, axis=2), axis=0)
        else:
            cand = jnp.where(keys < m_prev[:, None], keys, IMIN)
            m = jnp.max(cand, axis=1)
        vals = jnp.where(lane == j, m[:, None], vals)
        return m, vals

    m0 = jnp.full((nrows,), IMAX, jnp.int32)
    vals0 = jnp.full((nrows, k), IMIN, jnp.int32)
    _, vals = lax.fori_loop(0, k, it, (m0, vals0))
    return vals


def _prune(arr, k, depth):
    if depth == 0 or k == 1:
        return [(arr, k)]
    half = arr.shape[1] // 2
    a = arr[:, :half]
    b = arr[:, half:]
    return _prune(jnp.maximum(a, b), k, depth - 1) + _prune(
        jnp.minimum(a, b), k // 2, depth - 1
    )


PRUNE_DEPTH = 2
NCAND = 32 + 16 + 16 + 8


def _local_cands(keys, nrows):
    leaves = _prune(keys, K, PRUNE_DEPTH)
    return jnp.concatenate(
        [_topk_chain(arr, nrows, False, k) for arr, k in leaves], axis=1
    )


def kernel(x):
    m, n = x.shape
    mq = m // NQ

    def body(
        x_hbm,
        out_ref,
        xq_ref,
        commz_ref,
        copy_sem,
        z_send_sems,
        z_recv_sems,
        xy_send_sems,
        xy_recv_sems,
    ):
        my_x = lax.axis_index("x")
        my_y = lax.axis_index("y")
        my_z = lax.axis_index("z")
        q = my_x * 2 + my_y

        cp = pltpu.make_async_copy(
            x_hbm.at[pl.ds(q * mq, mq), :], xq_ref, copy_sem
        )
        cp.start()

        barrier = pltpu.get_barrier_semaphore()
        for dz in (1, 2, 3):
            tz = lax.rem(my_z + dz, NZ)
            pl.semaphore_signal(
                barrier, inc=1,
                device_id=(my_x, my_y, tz),
                device_id_type=pl.DeviceIdType.MESH,
            )
        for px, py in ((1, 0), (0, 1), (1, 1)):
            pl.semaphore_signal(
                barrier, inc=1,
                device_id=(my_x ^ px, my_y ^ py, my_z),
                device_id_type=pl.DeviceIdType.MESH,
            )
        pl.semaphore_wait(barrier, 6)

        cp.wait()

        keys = _encode(xq_ref[...], my_z)
        commz_ref[my_z, :, :] = _local_cands(keys, mq)

        z_sends = []
        for dz in (1, 2, 3):
            tz = lax.rem(my_z + dz, NZ)
            rdma = pltpu.make_async_remote_copy(
                src_ref=commz_ref.at[my_z],
                dst_ref=commz_ref.at[my_z],
                send_sem=z_send_sems.at[dz - 1],
                recv_sem=z_recv_sems.at[my_z],
                device_id=(my_x, my_y, tz),
                device_id_type=pl.DeviceIdType.MESH,
            )
            rdma.start()
            z_sends.append(rdma)
        for rdma in z_sends:
            rdma.wait_send()
        for dz in (1, 2, 3):
            sz = lax.rem(my_z + dz, NZ)
            recv = pltpu.make_async_remote_copy(
                src_ref=commz_ref.at[my_z],
                dst_ref=commz_ref.at[sz],
                send_sem=z_send_sems.at[0],
                recv_sem=z_recv_sems.at[sz],
                device_id=(my_x, my_y, sz),
                device_id_type=pl.DeviceIdType.MESH,
            )
            recv.wait_recv()

        gq = _topk_chain(commz_ref[...], mq, True)
        out_ref[pl.ds(q * mq, mq), :] = _decode(gq)

        xy_sends = []
        for i, (px, py) in enumerate(((1, 0), (0, 1), (1, 1))):
            rdma = pltpu.make_async_remote_copy(
                src_ref=out_ref.at[pl.ds(q * mq, mq), :],
                dst_ref=out_ref.at[pl.ds(q * mq, mq), :],
                send_sem=xy_send_sems.at[i],
                recv_sem=xy_recv_sems.at[q],
                device_id=(my_x ^ px, my_y ^ py, my_z),
                device_id_type=pl.DeviceIdType.MESH,
            )
            rdma.start()
            xy_sends.append(rdma)
        for rdma in xy_sends:
            rdma.wait_send()
        for px, py in ((1, 0), (0, 1), (1, 1)):
            qs = (my_x ^ px) * 2 + (my_y ^ py)
            recv = pltpu.make_async_remote_copy(
                src_ref=out_ref.at[pl.ds(q * mq, mq), :],
                dst_ref=out_ref.at[pl.ds(qs * mq, mq), :],
                send_sem=xy_send_sems.at[0],
                recv_sem=xy_recv_sems.at[qs],
                device_id=(my_x ^ px, my_y ^ py, my_z),
                device_id_type=pl.DeviceIdType.MESH,
            )
            recv.wait_recv()

    return pl.pallas_call(
        body,
        in_specs=[pl.BlockSpec(memory_space=pl.ANY)],
        out_specs=pl.BlockSpec(memory_space=pltpu.VMEM),
        out_shape=jax.ShapeDtypeStruct((m, K), jnp.float32),
        scratch_shapes=[
            pltpu.VMEM((m // NQ, n), jnp.float32),
            pltpu.VMEM((NZ, m // NQ, NCAND), jnp.int32),
            pltpu.SemaphoreType.DMA,
            pltpu.SemaphoreType.DMA((3,)),
            pltpu.SemaphoreType.DMA((NZ,)),
            pltpu.SemaphoreType.DMA((3,)),
            pltpu.SemaphoreType.DMA((NQ,)),
        ],
        compiler_params=pltpu.CompilerParams(
            collective_id=0,
            vmem_limit_bytes=64 * 1024 * 1024,
        ),
    )(x)
